# baseline (speedup 1.0000x reference)
import functools

import jax
import jax.numpy as jnp
from jax import lax
from jax.experimental import pallas as pl
from jax.experimental.pallas import tpu as pltpu

CV = 64
NP = 4
NO = 4


def _fused_body(nv_ref, cs_ref, ci_ref, ct_ref, x_hbm, w_ref, bias_ref,
                o_logits, o_k, buf, sems, *, B, V, D, O, cv, np_, no_):
    fc = D // np_
    bo = O // no_
    s = pl.program_id(0)
    T = ct_ref[0]

    def pool_range(p, r0, r1):
        fcol = pl.multiple_of(p * fc, 128)

        def chunk_src(t, slot):
            b = cs_ref[t]
            i = ci_ref[t]
            nv = jnp.minimum(nv_ref[b], V)
            last0 = jnp.maximum(0, ((nv - cv + 7) // 8) * 8)
            row0 = pl.multiple_of(jnp.minimum(i * cv, last0), 8)
            return x_hbm.at[b, pl.ds(row0, cv), pl.ds(fcol, fc)]

        def start(t, slot):
            pltpu.make_async_copy(chunk_src(t, slot), buf.at[slot],
                                  sems.at[slot]).start()

        @pl.when(r1 > r0)
        def _run():
            start(r0, 0)

            def step(t, carry):
                slot = lax.rem(t - r0, 2)

                @pl.when(t + 1 < r1)
                def _next():
                    start(t + 1, 1 - slot)

                pltpu.make_async_copy(
                    x_hbm.at[0, pl.ds(0, cv), pl.ds(0, fc)],
                    buf.at[slot], sems.at[slot]).wait()

                b = cs_ref[t]
                i = ci_ref[t]
                nv = jnp.minimum(nv_ref[b], V)
                last0 = jnp.maximum(0, ((nv - cv + 7) // 8) * 8)
                row0 = jnp.minimum(i * cv, last0)
                data = buf[slot]
                row = row0 + lax.broadcasted_iota(jnp.int32, (cv, 1), 0)
                data = jnp.where(row < nv, data, -jnp.inf)
                part = data[0:8]
                for r in range(1, cv // 8):
                    part = jnp.maximum(part, data[r * 8:(r + 1) * 8])
                m = jnp.max(part, axis=0, keepdims=True)

                @pl.when(i == 0)
                def _first():
                    o_k[pl.ds(b, 1), pl.ds(fcol, fc)] = m

                @pl.when(i > 0)
                def _more():
                    cur = o_k[pl.ds(b, 1), pl.ds(fcol, fc)]
                    o_k[pl.ds(b, 1), pl.ds(fcol, fc)] = jnp.maximum(cur, m)

                return carry

            lax.fori_loop(r0, r1, step, 0)

    @pl.when(s == 0)
    def _lead_in():
        pool_range(0, 0, T)

    @pl.when(s > 0)
    def _steady():
        c = (s - 1) // no_
        o = lax.rem(s - 1, no_)

        @pl.when(c + 1 < np_)
        def _pool_next():
            u = lax.rem(s - 1, no_)
            q = (T + no_ - 1) // no_
            pool_range(c + 1, jnp.minimum(u * q, T),
                       jnp.minimum((u + 1) * q, T))

        kblk = o_k[:, pl.ds(pl.multiple_of(c * fc, 128), fc)]
        partial = lax.dot_general(
            kblk, w_ref[...],
            dimension_numbers=(((1,), (1,)), ((), ())),
            preferred_element_type=jnp.float32,
        )
        osl = pl.ds(pl.multiple_of(o * bo, 128), bo)

        @pl.when(c == 0)
        def _init():
            o_logits[:, osl] = partial + bias_ref[:, osl]

        @pl.when(c > 0)
        def _acc():
            o_logits[:, osl] = o_logits[:, osl] + partial


def kernel(batch_size, max_num_views, num_views, x, W, b):
    B, V, D = x.shape
    O = W.shape[0]
    fc = D // NP
    bo = O // NO

    nv = jnp.minimum(num_views.astype(jnp.int32), V)
    counts = (nv + CV - 1) // CV
    T = jnp.sum(counts)
    ends = jnp.cumsum(counts)
    starts = ends - counts
    t_idx = jnp.arange(B * (V // CV), dtype=jnp.int32)
    chunk_sample = (jnp.searchsorted(ends, t_idx, side="right")
                    .astype(jnp.int32))
    chunk_sample = jnp.minimum(chunk_sample, B - 1)
    chunk_idx = t_idx - starts[chunk_sample]

    bias = jnp.broadcast_to(b.reshape(1, O), (B, O))

    def w_index(s, *_):
        c = jnp.maximum(s - 1, 0) // NO
        o = lax.rem(jnp.maximum(s - 1, 0), NO)
        return o, c

    fused = pl.pallas_call(
        functools.partial(_fused_body, B=B, V=V, D=D, O=O,
                          cv=CV, np_=NP, no_=NO),
        grid_spec=pltpu.PrefetchScalarGridSpec(
            num_scalar_prefetch=4,
            grid=(1 + NP * NO,),
            in_specs=[
                pl.BlockSpec(memory_space=pl.ANY),
                pl.BlockSpec((bo, fc), w_index),
                pl.BlockSpec((B, O), lambda s, *_: (0, 0)),
            ],
            out_specs=[
                pl.BlockSpec((B, O), lambda s, *_: (0, 0)),
                pl.BlockSpec((B, D), lambda s, *_: (0, 0)),
            ],
            scratch_shapes=[
                pltpu.VMEM((2, CV, fc), jnp.float32),
                pltpu.SemaphoreType.DMA((2,)),
            ],
        ),
        out_shape=[
            jax.ShapeDtypeStruct((B, O), jnp.float32),
            jax.ShapeDtypeStruct((B, D), jnp.float32),
        ],
        compiler_params=pltpu.CompilerParams(
            dimension_semantics=("arbitrary",),
        ),
    )
    logits, k = fused(nv, chunk_sample, chunk_idx,
                      T.reshape(1), x, W, bias)
    return (logits, k)


# speedup vs baseline: 3.3253x; 3.3253x over previous
import functools

import jax
import jax.numpy as jnp
from jax import lax
from jax.experimental import pallas as pl
from jax.experimental.pallas import tpu as pltpu

BV = 256
BO = 512


def _pool_body(nv_ref, x_ref, o_ref, *, bv, max_views):
    b = pl.program_id(0)
    j = pl.program_id(1)
    nv = jnp.minimum(nv_ref[b], max_views)
    jmax = (nv + bv - 1) // bv - 1

    @pl.when(j == 0)
    def _init():
        o_ref[...] = jnp.full_like(o_ref, -jnp.inf)

    @pl.when(j <= jmax)
    def _update():
        jb = jnp.minimum(j, jmax)
        row = jb * bv + lax.broadcasted_iota(jnp.int32, (bv, 1), 0)
        blk = jnp.where(row < nv, x_ref[0], -jnp.inf)
        part = blk[0:8]
        for r in range(1, bv // 8):
            part = jnp.maximum(part, blk[r * 8:(r + 1) * 8])
        o_ref[0] = jnp.maximum(o_ref[0], jnp.max(part, axis=0, keepdims=True))


def _linear_body(k_ref, w_ref, bias_ref, o_ref):
    out = lax.dot_general(
        k_ref[...], w_ref[...],
        dimension_numbers=(((1,), (1,)), ((), ())),
        preferred_element_type=jnp.float32,
    )
    o_ref[...] = out + bias_ref[...]


def kernel(batch_size, max_num_views, num_views, x, W, b):
    B, V, D = x.shape
    O = W.shape[0]

    def x_index(bi, j, nv_ref):
        nv = jnp.minimum(nv_ref[bi], V)
        jmax = (nv + BV - 1) // BV - 1
        return bi, jnp.minimum(j, jmax), 0

    pool = pl.pallas_call(
        functools.partial(_pool_body, bv=BV, max_views=V),
        grid_spec=pltpu.PrefetchScalarGridSpec(
            num_scalar_prefetch=1,
            grid=(B, V // BV),
            in_specs=[pl.BlockSpec((1, BV, D), x_index)],
            out_specs=pl.BlockSpec((1, 1, D), lambda bi, j, nv_ref: (bi, 0, 0)),
        ),
        out_shape=jax.ShapeDtypeStruct((B, 1, D), jnp.float32),
        compiler_params=pltpu.CompilerParams(
            dimension_semantics=("arbitrary", "arbitrary"),
        ),
    )
    k = pool(num_views.astype(jnp.int32), x).reshape(B, D)

    bias = b.reshape(1, O)
    linear = pl.pallas_call(
        _linear_body,
        grid=(O // BO,),
        in_specs=[
            pl.BlockSpec((B, D), lambda o: (0, 0)),
            pl.BlockSpec((BO, D), lambda o: (o, 0)),
            pl.BlockSpec((1, BO), lambda o: (0, o)),
        ],
        out_specs=pl.BlockSpec((B, BO), lambda o: (0, o)),
        out_shape=jax.ShapeDtypeStruct((B, O), jnp.float32),
        compiler_params=pltpu.CompilerParams(
            dimension_semantics=("arbitrary",),
        ),
    )
    logits = linear(k, W, bias)
    return (logits, k)
